# baseline (speedup 1.0000x reference)
import functools

import numpy as np
import jax
import jax.numpy as jnp
from jax import lax
from jax.experimental import pallas as pl
from jax.experimental.pallas import tpu as pltpu
from jax.experimental.pallas import tpu_sc as plsc

D_MODEL = 768
MAX_SEQ_LEN = 4096
BATCH = 4
SEQ_LEN = 4096
N_ROWS = BATCH * SEQ_LEN

NUM_CORES = 2
NUM_SUBCORES = 16
LANES = 16
NUM_WORKERS = NUM_CORES * NUM_SUBCORES
POS_PER_WORKER = SEQ_LEN // NUM_WORKERS
CHUNK = 16
NUM_CHUNKS = POS_PER_WORKER // CHUNK
NBUF = 2

SCALE = float(np.sqrt(np.float32(D_MODEL)))


def _sinusoidal_pe_np(max_seq_len, d_model):
    position = np.arange(0, max_seq_len, dtype=np.float32)[:, None]
    div_term = np.exp(
        np.arange(0, d_model, 2).astype(np.float32) * (-np.log(10000.0) / d_model)
    )
    pe = np.zeros((max_seq_len, d_model), dtype=np.float32)
    pe[:, 0::2] = np.sin(position * div_term)
    pe[:, 1::2] = np.cos(position * div_term)
    return pe


_PE = _sinusoidal_pe_np(MAX_SEQ_LEN, D_MODEL)


_MESH = plsc.VectorSubcoreMesh(core_axis_name="c", subcore_axis_name="s")


@functools.partial(
    pl.kernel,
    mesh=_MESH,
    out_type=jax.ShapeDtypeStruct((N_ROWS, D_MODEL), jnp.float32),
    scratch_types=[
        pltpu.VMEM((BATCH, POS_PER_WORKER), jnp.int32),
        pltpu.VMEM((NBUF, BATCH, CHUNK, D_MODEL), jnp.float32),
        pltpu.VMEM((NBUF, CHUNK, D_MODEL), jnp.float32),
        pltpu.SemaphoreType.DMA,
        pltpu.SemaphoreType.DMA,
        pltpu.SemaphoreType.DMA,
    ],
)
def _embed_sc(x_hbm, table_hbm, pe_hbm, out_hbm,
              idx_v, rows_v, pe_v, gsem, psem, osem):
    wid = lax.axis_index("s") * NUM_CORES + lax.axis_index("c")
    pos0 = wid * POS_PER_WORKER

    for b in range(BATCH):
        pltpu.sync_copy(
            x_hbm.at[b, pl.ds(pos0, POS_PER_WORKER)], idx_v.at[b])

    def gather_copy(g, buf, b):
        return pltpu.make_async_copy(
            table_hbm.at[idx_v.at[b, pl.ds(g * CHUNK, CHUNK)]],
            rows_v.at[buf, b], gsem)

    def pe_copy(g, buf):
        return pltpu.make_async_copy(
            pe_hbm.at[pl.ds(pos0 + g * CHUNK, CHUNK)], pe_v.at[buf], psem)

    def out_copy(g, buf, b):
        return pltpu.make_async_copy(
            rows_v.at[buf, b],
            out_hbm.at[pl.ds(b * SEQ_LEN + pos0 + g * CHUNK, CHUNK)], osem)

    for g in range(NBUF - 1):
        for b in range(BATCH):
            gather_copy(g, g, b).start()
        pe_copy(g, g).start()

    def chunk_body(g, carry):
        buf = g % NBUF
        with jax.named_scope("gwait"):
            for b in range(BATCH):
                gather_copy(g, buf, b).wait()
            pe_copy(g, buf).wait()

        with jax.named_scope("prefetch"):
            @pl.when(g < NUM_CHUNKS - (NBUF - 1))
            def _():
                nxt = (g + NBUF - 1) % NBUF
                @pl.when(g >= 1)
                def _():
                    for b in range(BATCH):
                        out_copy(0, 0, 0).wait()
                for b in range(BATCH):
                    gather_copy(g + NBUF - 1, nxt, b).start()
                pe_copy(g + NBUF - 1, nxt).start()

        with jax.named_scope("fma"):
            @plsc.parallel_loop(0, CHUNK, 1, unroll=2)
            def _(r):
                for j in range(D_MODEL // LANES):
                    sl = pl.ds(j * LANES, LANES)
                    pv = pe_v[buf, r, sl]
                    for b in range(BATCH):
                        rows_v[buf, b, r, sl] = (
                            pv + rows_v[buf, b, r, sl] * SCALE)

        with jax.named_scope("ostart"):
            for b in range(BATCH):
                out_copy(g, buf, b).start()
        return carry

    lax.fori_loop(0, NUM_CHUNKS, chunk_body, 0, unroll=2)

    for _ in range(NBUF * BATCH):
        out_copy(0, 0, 0).wait()


def kernel(x, table):
    xf = x.astype(jnp.int32)
    pe = jnp.asarray(_PE)
    out = _embed_sc(xf, table, pe)
    return out.reshape(BATCH, SEQ_LEN, D_MODEL)


# speedup vs baseline: 1.1081x; 1.1081x over previous
import functools

import numpy as np
import jax
import jax.numpy as jnp
from jax import lax
from jax.experimental import pallas as pl
from jax.experimental.pallas import tpu as pltpu
from jax.experimental.pallas import tpu_sc as plsc

D_MODEL = 768
MAX_SEQ_LEN = 4096
BATCH = 4
SEQ_LEN = 4096
N_ROWS = BATCH * SEQ_LEN

NUM_CORES = 2
NUM_SUBCORES = 16
LANES = 16
NUM_WORKERS = NUM_CORES * NUM_SUBCORES
POS_PER_WORKER = SEQ_LEN // NUM_WORKERS
CHUNK = 16
NUM_CHUNKS = POS_PER_WORKER // CHUNK
NBUF = 2

SCALE = float(np.sqrt(np.float32(D_MODEL)))


def _sinusoidal_pe_np(max_seq_len, d_model):
    position = np.arange(0, max_seq_len, dtype=np.float32)[:, None]
    div_term = np.exp(
        np.arange(0, d_model, 2).astype(np.float32) * (-np.log(10000.0) / d_model)
    )
    pe = np.zeros((max_seq_len, d_model), dtype=np.float32)
    pe[:, 0::2] = np.sin(position * div_term)
    pe[:, 1::2] = np.cos(position * div_term)
    return pe


_PE = _sinusoidal_pe_np(MAX_SEQ_LEN, D_MODEL)


_MESH = plsc.VectorSubcoreMesh(core_axis_name="c", subcore_axis_name="s")


@functools.partial(
    pl.kernel,
    mesh=_MESH,
    out_type=jax.ShapeDtypeStruct((N_ROWS, D_MODEL), jnp.float32),
    scratch_types=[
        pltpu.VMEM((BATCH, POS_PER_WORKER), jnp.int32),
        pltpu.VMEM((NBUF, BATCH, CHUNK, D_MODEL), jnp.float32),
        pltpu.VMEM((NBUF, CHUNK, D_MODEL), jnp.float32),
        pltpu.SemaphoreType.DMA,
        pltpu.SemaphoreType.DMA,
        pltpu.SemaphoreType.DMA,
    ],
)
def _embed_sc(x_hbm, table_hbm, pe_hbm, out_hbm,
              idx_v, rows_v, pe_v, gsem, psem, osem):
    wid = lax.axis_index("s") * NUM_CORES + lax.axis_index("c")
    pos0 = wid * POS_PER_WORKER

    for b in range(BATCH):
        pltpu.sync_copy(
            x_hbm.at[b, pl.ds(pos0, POS_PER_WORKER)], idx_v.at[b])

    def gather_copy(g, buf, b):
        return pltpu.make_async_copy(
            table_hbm.at[idx_v.at[b, pl.ds(g * CHUNK, CHUNK)]],
            rows_v.at[buf, b], gsem)

    def pe_copy(g, buf):
        return pltpu.make_async_copy(
            pe_hbm.at[pl.ds(pos0 + g * CHUNK, CHUNK)], pe_v.at[buf], psem)

    def out_copy(g, buf, b):
        return pltpu.make_async_copy(
            rows_v.at[buf, b],
            out_hbm.at[pl.ds(b * SEQ_LEN + pos0 + g * CHUNK, CHUNK)], osem)

    for g in range(NBUF - 1):
        for b in range(BATCH):
            gather_copy(g, g, b).start()
        pe_copy(g, g).start()

    def chunk_body(g, carry):
        buf = g % NBUF
        with jax.named_scope("gwait"):
            for b in range(BATCH):
                gather_copy(g, buf, b).wait()
            pe_copy(g, buf).wait()

        with jax.named_scope("prefetch"):
            @pl.when(g < NUM_CHUNKS - (NBUF - 1))
            def _():
                nxt = (g + NBUF - 1) % NBUF
                @pl.when(g >= 1)
                def _():
                    for b in range(BATCH):
                        out_copy(0, 0, 0).wait()
                for b in range(BATCH):
                    gather_copy(g + NBUF - 1, nxt, b).start()
                pe_copy(g + NBUF - 1, nxt).start()

        with jax.named_scope("fma"):
            @plsc.parallel_loop(0, CHUNK, 1, unroll=3)
            def _(r):
                for j in range(D_MODEL // LANES):
                    sl = pl.ds(j * LANES, LANES)
                    pv = pe_v[buf, r, sl]
                    for b in range(BATCH):
                        rows_v[buf, b, r, sl] = (
                            pv + rows_v[buf, b, r, sl] * SCALE)

        with jax.named_scope("ostart"):
            for b in range(BATCH):
                out_copy(g, buf, b).start()
        return carry

    lax.fori_loop(0, NUM_CHUNKS, chunk_body, 0)

    for _ in range(NBUF * BATCH):
        out_copy(0, 0, 0).wait()


def kernel(x, table):
    xf = x.astype(jnp.int32)
    pe = jnp.asarray(_PE)
    out = _embed_sc(xf, table, pe)
    return out.reshape(BATCH, SEQ_LEN, D_MODEL)


# speedup vs baseline: 3.2290x; 2.9141x over previous
import functools

import numpy as np
import jax
import jax.numpy as jnp
from jax import lax
from jax.experimental import pallas as pl
from jax.experimental.pallas import tpu as pltpu
from jax.experimental.pallas import tpu_sc as plsc

D_MODEL = 768
MAX_SEQ_LEN = 4096
BATCH = 4
SEQ_LEN = 4096
N_ROWS = BATCH * SEQ_LEN

NUM_CORES = 2
NUM_SUBCORES = 16
LANES = 16
NUM_WORKERS = NUM_CORES * NUM_SUBCORES
POS_PER_WORKER = SEQ_LEN // NUM_WORKERS
CHUNK = 16
NUM_CHUNKS = POS_PER_WORKER // CHUNK
NBUF = 2

SCALE = float(np.sqrt(np.float32(D_MODEL)))


def _sinusoidal_pe_np(max_seq_len, d_model):
    position = np.arange(0, max_seq_len, dtype=np.float32)[:, None]
    div_term = np.exp(
        np.arange(0, d_model, 2).astype(np.float32) * (-np.log(10000.0) / d_model)
    )
    pe = np.zeros((max_seq_len, d_model), dtype=np.float32)
    pe[:, 0::2] = np.sin(position * div_term)
    pe[:, 1::2] = np.cos(position * div_term)
    return pe


_PE = _sinusoidal_pe_np(MAX_SEQ_LEN, D_MODEL)


_MESH = plsc.VectorSubcoreMesh(core_axis_name="c", subcore_axis_name="s")


@functools.partial(
    pl.kernel,
    mesh=_MESH,
    out_type=jax.ShapeDtypeStruct((N_ROWS, D_MODEL), jnp.float32),
    scratch_types=[
        pltpu.VMEM((BATCH, POS_PER_WORKER), jnp.int32),
        pltpu.VMEM((NBUF, BATCH, CHUNK, D_MODEL), jnp.float32),
        pltpu.VMEM((NBUF, CHUNK, D_MODEL), jnp.float32),
        pltpu.SemaphoreType.DMA,
        pltpu.SemaphoreType.DMA,
        pltpu.SemaphoreType.DMA,
    ],
)
def _embed_sc(x_hbm, table_hbm, pe_hbm, out_hbm,
              idx_v, rows_v, pe_v, gsem, psem, osem):
    wid = lax.axis_index("s") * NUM_CORES + lax.axis_index("c")
    pos0 = wid * POS_PER_WORKER

    for b in range(BATCH):
        pltpu.sync_copy(
            x_hbm.at[b, pl.ds(pos0, POS_PER_WORKER)], idx_v.at[b])

    def gather_copy(g, buf, b):
        return pltpu.make_async_copy(
            table_hbm.at[idx_v.at[b, pl.ds(g * CHUNK, CHUNK)]],
            rows_v.at[buf, b], gsem)

    def pe_copy(g, buf):
        return pltpu.make_async_copy(
            pe_hbm.at[pl.ds(pos0 + g * CHUNK, CHUNK)], pe_v.at[buf], psem)

    def out_copy(g, buf, b):
        return pltpu.make_async_copy(
            rows_v.at[buf, b],
            out_hbm.at[pl.ds(b * SEQ_LEN + pos0 + g * CHUNK, CHUNK)], osem)

    for g in range(NBUF - 1):
        for b in range(BATCH):
            gather_copy(g, g, b).start()
        pe_copy(g, g).start()

    def chunk_body(g, carry):
        buf = g % NBUF
        with jax.named_scope("gwait"):
            for b in range(BATCH):
                gather_copy(g, buf, b).wait()
            pe_copy(g, buf).wait()

        with jax.named_scope("prefetch"):
            @pl.when(g < NUM_CHUNKS - (NBUF - 1))
            def _():
                nxt = (g + NBUF - 1) % NBUF
                @pl.when(g >= 1)
                def _():
                    for b in range(BATCH):
                        out_copy(0, 0, 0).wait()
                for b in range(BATCH):
                    gather_copy(g + NBUF - 1, nxt, b).start()
                pe_copy(g + NBUF - 1, nxt).start()

        with jax.named_scope("fma"):
            @plsc.parallel_loop(0, CHUNK, 1, unroll=2)
            def _(r):
                for j in range(D_MODEL // LANES):
                    sl = pl.ds(j * LANES, LANES)
                    pv = pe_v[buf, r, sl]
                    for b in range(BATCH):
                        rows_v[buf, b, r, sl] = (
                            pv + rows_v[buf, b, r, sl] * SCALE)

        with jax.named_scope("ostart"):
            for b in range(BATCH):
                out_copy(g, buf, b).start()
        return carry

    lax.fori_loop(0, NUM_CHUNKS, chunk_body, 0)

    for _ in range(NBUF * BATCH):
        out_copy(0, 0, 0).wait()


def kernel(x, table):
    xf = x.astype(jnp.int32)
    pe = jnp.asarray(_PE)
    out = _embed_sc(xf, table, pe)
    return out.reshape(BATCH, SEQ_LEN, D_MODEL)
